# baseline (speedup 1.0000x reference)
import functools

import jax
import jax.numpy as jnp
from jax import lax
from jax.experimental import pallas as pl
from jax.experimental.pallas import tpu as pltpu
from jax.experimental.pallas import tpu_sc as plsc

_MAX_CLASSES = 1000000
_D = 16
_B, _N = 16384, 200

_info = plsc.get_sparse_core_info()
_NC, _NS, _L = _info.num_cores, _info.num_subcores, _info.num_lanes
_NW = _NC * _NS
_W = 2048
_WINS_PER_N = _B // _W
_TASKS = _N * _WINS_PER_N
_PER_W = _TASKS // _NW


@functools.partial(
    pl.kernel,
    out_type=jax.ShapeDtypeStruct((_N, _D, _B), jnp.float32),
    mesh=plsc.VectorSubcoreMesh(core_axis_name="c", subcore_axis_name="s"),
    scratch_types=[
        pltpu.VMEM((2, _W // 128, 128), jnp.int32),
        pltpu.VMEM((2, _W, _D), jnp.float32),
        pltpu.VMEM((_D, _W), jnp.float32),
        pltpu.SemaphoreType.DMA((2,)),
        pltpu.SemaphoreType.DMA,
    ],
    compiler_params=pltpu.CompilerParams(
        use_tc_tiling_on_sc=False, needs_layout_passes=False
    ),
)
def _gather_kernel(labels_hbm, table_hbm, out_hbm, idx_v, rows_v, tr_v, gsem, ssem):
    wid = lax.axis_index("s") * _NC + lax.axis_index("c")
    t0 = wid * _PER_W

    def task_nb(t):
        n = t // _WINS_PER_N
        b0 = (t % _WINS_PER_N) * _W
        return n, b0

    def load_and_fire(t, b):
        n, b0 = task_nb(t)
        tr = n // 8
        ir = n % 8
        tc0 = b0 // 128
        pltpu.sync_copy(
            labels_hbm.at[pl.ds(tr * 128 + tc0, _W // 128), pl.ds(ir * 128, 128)],
            idx_v.at[b],
        )

        def remap(r, c):
            for cc in range(128 // _L):
                v = idx_v[b, r, pl.ds(cc * _L, _L)]
                w = jnp.minimum(jnp.maximum(v, 0), _MAX_CLASSES)
                idx_v[b, r, pl.ds(cc * _L, _L)] = jnp.where(v == -1, _MAX_CLASSES, w)
            return c

        lax.fori_loop(0, _W // 128, remap, 0)

        def gfire(r, c):
            pltpu.async_copy(
                table_hbm.at[idx_v.at[b, r]],
                rows_v.at[b, pl.ds(r * 128, 128)],
                gsem.at[b],
            )
            return c

        lax.fori_loop(0, _W // 128, gfire, 0)

    def wait_gather(b):
        def gwait(r, c):
            pltpu.make_async_copy(
                table_hbm.at[idx_v.at[b, r]],
                rows_v.at[b, pl.ds(r * 128, 128)],
                gsem.at[b],
            ).wait()
            return c

        lax.fori_loop(0, _W // 128, gwait, 0)

    def transpose(b):
        iota = lax.iota(jnp.int32, _L)
        dconst = [jnp.full((_L,), d, jnp.int32) for d in range(_D)]

        def body(c, r_ids):
            vals = [
                plsc.load_gather(rows_v.at[b], [r_ids, dconst[d]])
                for d in range(_D)
            ]
            for d in range(_D):
                tr_v[d, pl.ds(c * _L, _L)] = vals[d]
            return r_ids + _L

        lax.fori_loop(0, _W // _L, body, iota, unroll=2)

    def fire_store(t):
        n, b0 = task_nb(t)
        pltpu.async_copy(tr_v, out_hbm.at[n, :, pl.ds(b0, _W)], ssem)

    def wait_store(t):
        n, b0 = task_nb(t)
        pltpu.make_async_copy(tr_v, out_hbm.at[n, :, pl.ds(b0, _W)], ssem).wait()

    load_and_fire(t0, 0)

    def step(i, carry):
        t = t0 + i
        b = i % 2

        @pl.when(i < _PER_W - 1)
        def _():
            load_and_fire(t + 1, 1 - b)

        wait_gather(b)

        @pl.when(i >= 1)
        def _():
            wait_store(t - 1)

        transpose(b)
        fire_store(t)
        return carry

    lax.fori_loop(0, _PER_W, step, 0)
    wait_store(t0 + _PER_W - 1)


def kernel(labels, table):
    lt2 = (
        labels.T.reshape(_N // 8, 8, _B // 128, 128)
        .transpose(0, 2, 1, 3)
        .reshape((_N // 8) * (_B // 128), 8 * 128)
    )
    eye = lax.optimization_barrier(jnp.eye(_D, dtype=jnp.float32))
    trm = jax.lax.dot_general(
        table.T, eye, (((0,), (0,)), ((), ())),
        preferred_element_type=jnp.float32,
    )
    out_t = _gather_kernel(lt2, trm)
    return jnp.transpose(out_t, (2, 0, 1))


# speedup vs baseline: 1.0106x; 1.0106x over previous
import functools

import jax
import jax.numpy as jnp
from jax import lax
from jax.experimental import pallas as pl
from jax.experimental.pallas import tpu as pltpu
from jax.experimental.pallas import tpu_sc as plsc

_MAX_CLASSES = 1000000
_D = 16
_B, _N = 16384, 200

_info = plsc.get_sparse_core_info()
_NC, _NS, _L = _info.num_cores, _info.num_subcores, _info.num_lanes
_NW = _NC * _NS
_W = 2048
_WINS_PER_N = _B // _W
_TASKS = _N * _WINS_PER_N
_PER_W = _TASKS // _NW

_VPAD = 1000064
_NTC = _VPAD // 128
_P1_PER_TILE = 489
_P1_CHUNKS = 245


@functools.partial(
    pl.kernel,
    out_type=(
        jax.ShapeDtypeStruct((_N, _D, _B), jnp.float32),
        jax.ShapeDtypeStruct((_NC * _VPAD, _D), jnp.float32),
    ),
    mesh=plsc.VectorSubcoreMesh(core_axis_name="c", subcore_axis_name="s"),
    scratch_types=[
        pltpu.VMEM((2, _W // 128, 128), jnp.int32),
        pltpu.VMEM((2, _W, _D), jnp.float32),
        pltpu.VMEM((_D, _W), jnp.float32),
        pltpu.VMEM((2, 32, 128), jnp.float32),
        pltpu.VMEM((2, 256, _D), jnp.float32),
        pltpu.SemaphoreType.DMA((2,)),
        pltpu.SemaphoreType.DMA,
        pltpu.SemaphoreType.DMA((2,)),
        pltpu.SemaphoreType.DMA((2,)),
    ],
    compiler_params=pltpu.CompilerParams(
        use_tc_tiling_on_sc=False, needs_layout_passes=False
    ),
)
def _gather_kernel(
    labels_hbm, t3_hbm, out_hbm, tbl_hbm,
    idx_v, rows_v, tr_v, stage_v, ost_v, gsem, ssem, isem, psem,
):
    core = lax.axis_index("c")
    sub = lax.axis_index("s")
    wid = sub * _NC + core
    t0 = wid * _PER_W
    core_row0 = core * _VPAD

    tc_lo = sub * _P1_PER_TILE
    io16 = lax.iota(jnp.int32, _L)
    rv = [(io16 // 8) * 16 + tcrel * 8 + (io16 % 8) for tcrel in range(2)]

    def p1_tc0(i):
        return jnp.minimum(tc_lo + 2 * i, _NTC - 2)

    def p1_load(i, pb):
        tc0 = p1_tc0(i)
        pltpu.async_copy(
            t3_hbm.at[0, pl.ds(tc0 * 8, 16)], stage_v.at[pb, pl.ds(0, 16)],
            isem.at[pb],
        )
        pltpu.async_copy(
            t3_hbm.at[1, pl.ds(tc0 * 8, 16)], stage_v.at[pb, pl.ds(16, 16)],
            isem.at[pb],
        )

    def p1_wait_load(i, pb):
        tc0 = p1_tc0(i)
        pltpu.make_async_copy(
            t3_hbm.at[0, pl.ds(tc0 * 8, 16)], stage_v.at[pb, pl.ds(0, 16)],
            isem.at[pb],
        ).wait()
        pltpu.make_async_copy(
            t3_hbm.at[1, pl.ds(tc0 * 8, 16)], stage_v.at[pb, pl.ds(16, 16)],
            isem.at[pb],
        ).wait()

    def p1_store_slice(i):
        tc0 = p1_tc0(i)
        return tbl_hbm.at[pl.ds(core_row0 + tc0 * 128, 256)]

    def p1_compute_fire(i, pb):
        zero16 = jnp.zeros((_L,), jnp.int32)
        for tcrel in range(2):
            rvec = rv[tcrel]

            def icg_body(g, c):
                icb = zero16 + g * _L
                vals = [
                    plsc.load_gather(stage_v.at[pb], [rvec, icb + k])
                    for k in range(_L)
                ]
                for k in range(_L):
                    ost_v[pb, tcrel * 128 + g * _L + k, :] = vals[k]
                return c

            lax.fori_loop(0, 128 // _L, icg_body, 0)
        pltpu.async_copy(ost_v.at[pb], p1_store_slice(i), psem.at[pb])

    def p1_wait_store(i, pb):
        pltpu.make_async_copy(ost_v.at[pb], p1_store_slice(i), psem.at[pb]).wait()

    p1_load(0, 0)

    def p1_step(i, c):
        pb = i % 2

        @pl.when(i + 1 < _P1_CHUNKS)
        def _():
            p1_load(i + 1, 1 - pb)

        p1_wait_load(i, pb)

        @pl.when(i >= 2)
        def _():
            p1_wait_store(i - 2, pb)

        p1_compute_fire(i, pb)
        return c

    lax.fori_loop(0, _P1_CHUNKS, p1_step, 0)
    p1_wait_store(_P1_CHUNKS - 2, (_P1_CHUNKS - 2) % 2)
    p1_wait_store(_P1_CHUNKS - 1, (_P1_CHUNKS - 1) % 2)
    plsc.subcore_barrier()

    def task_nb(t):
        n = t // _WINS_PER_N
        b0 = (t % _WINS_PER_N) * _W
        return n, b0

    def load_and_fire(t, b):
        n, b0 = task_nb(t)
        tr = n // 8
        ir = n % 8
        tc0 = b0 // 128
        pltpu.sync_copy(
            labels_hbm.at[pl.ds(tr * 128 + tc0, _W // 128), pl.ds(ir * 128, 128)],
            idx_v.at[b],
        )
        coff = jnp.zeros((_L,), jnp.int32) + core_row0

        def remap(r, c):
            for cc in range(128 // _L):
                v = idx_v[b, r, pl.ds(cc * _L, _L)]
                w = jnp.minimum(jnp.maximum(v, 0), _MAX_CLASSES)
                idx_v[b, r, pl.ds(cc * _L, _L)] = (
                    jnp.where(v == -1, _MAX_CLASSES, w) + coff
                )
            return c

        lax.fori_loop(0, _W // 128, remap, 0)

        def gfire(r, c):
            pltpu.async_copy(
                tbl_hbm.at[idx_v.at[b, r]],
                rows_v.at[b, pl.ds(r * 128, 128)],
                gsem.at[b],
            )
            return c

        lax.fori_loop(0, _W // 128, gfire, 0)

    def wait_gather(b):
        def gwait(r, c):
            pltpu.make_async_copy(
                tbl_hbm.at[idx_v.at[b, r]],
                rows_v.at[b, pl.ds(r * 128, 128)],
                gsem.at[b],
            ).wait()
            return c

        lax.fori_loop(0, _W // 128, gwait, 0)

    def transpose(b):
        iota = lax.iota(jnp.int32, _L)
        dconst = [jnp.full((_L,), d, jnp.int32) for d in range(_D)]

        def body(c, r_ids):
            vals = [
                plsc.load_gather(rows_v.at[b], [r_ids, dconst[d]])
                for d in range(_D)
            ]
            for d in range(_D):
                tr_v[d, pl.ds(c * _L, _L)] = vals[d]
            return r_ids + _L

        lax.fori_loop(0, _W // _L, body, iota, unroll=2)

    def fire_store(t):
        n, b0 = task_nb(t)
        pltpu.async_copy(tr_v, out_hbm.at[n, :, pl.ds(b0, _W)], ssem)

    def wait_store(t):
        n, b0 = task_nb(t)
        pltpu.make_async_copy(tr_v, out_hbm.at[n, :, pl.ds(b0, _W)], ssem).wait()

    load_and_fire(t0, 0)

    def step(i, carry):
        t = t0 + i
        b = i % 2

        @pl.when(i < _PER_W - 1)
        def _():
            load_and_fire(t + 1, 1 - b)

        wait_gather(b)

        @pl.when(i >= 1)
        def _():
            wait_store(t - 1)

        transpose(b)
        fire_store(t)
        return carry

    lax.fori_loop(0, _PER_W, step, 0)
    wait_store(t0 + _PER_W - 1)


def kernel(labels, table):
    lt2 = (
        labels.T.reshape(_N // 8, 8, _B // 128, 128)
        .transpose(0, 2, 1, 3)
        .reshape((_N // 8) * (_B // 128), 8 * 128)
    )
    tp = jnp.pad(table, ((0, _VPAD - (_MAX_CLASSES + 1)), (0, 0)))
    t3 = (
        tp.T.reshape(2, 8, _NTC, 128)
        .transpose(0, 2, 1, 3)
        .reshape(2, _NTC * 8, 128)
    )
    out_t, _ = _gather_kernel(lt2, t3)
    return jnp.transpose(out_t, (2, 0, 1))


# speedup vs baseline: 1.2680x; 1.2547x over previous
import functools

import jax
import jax.numpy as jnp
from jax import lax
from jax.experimental import pallas as pl
from jax.experimental.pallas import tpu as pltpu
from jax.experimental.pallas import tpu_sc as plsc

_MAX_CLASSES = 1000000
_D = 16
_B, _N = 16384, 200

_info = plsc.get_sparse_core_info()
_NC, _NS, _L = _info.num_cores, _info.num_subcores, _info.num_lanes
_NW = _NC * _NS
_W = 2048
_WINS_PER_N = _B // _W
_TASKS = _N * _WINS_PER_N
_PER_W = _TASKS // _NW

_VPAD = 1000064
_NTC = _VPAD // 128
_P1_PER_TILE = 489
_P1_CHUNKS = 245


@functools.partial(
    pl.kernel,
    out_type=(
        jax.ShapeDtypeStruct((_N, 2, _B // 128, 8, 128), jnp.float32),
        jax.ShapeDtypeStruct((_NC * _VPAD, _D), jnp.float32),
    ),
    mesh=plsc.VectorSubcoreMesh(core_axis_name="c", subcore_axis_name="s"),
    scratch_types=[
        pltpu.VMEM((2, _W // 128, 128), jnp.int32),
        pltpu.VMEM((2, _W, _D), jnp.float32),
        pltpu.VMEM((2, _W // 128, 8, 128), jnp.float32),
        pltpu.VMEM((2, 32, 128), jnp.float32),
        pltpu.VMEM((2, 256, _D), jnp.float32),
        pltpu.SemaphoreType.DMA((2,)),
        pltpu.SemaphoreType.DMA,
        pltpu.SemaphoreType.DMA((2,)),
        pltpu.SemaphoreType.DMA((2,)),
    ],
    compiler_params=pltpu.CompilerParams(
        use_tc_tiling_on_sc=False, needs_layout_passes=False
    ),
)
def _gather_kernel(
    labels_hbm, t3_hbm, out_hbm, tbl_hbm,
    idx_v, rows_v, tr_v, stage_v, ost_v, gsem, ssem, isem, psem,
):
    core = lax.axis_index("c")
    sub = lax.axis_index("s")
    wid = sub * _NC + core
    t0 = wid * _PER_W
    core_row0 = core * _VPAD

    tc_lo = sub * _P1_PER_TILE
    io16 = lax.iota(jnp.int32, _L)
    rv = [(io16 // 8) * 16 + tcrel * 8 + (io16 % 8) for tcrel in range(2)]

    def p1_tc0(i):
        return jnp.minimum(tc_lo + 2 * i, _NTC - 2)

    def p1_load(i, pb):
        tc0 = p1_tc0(i)
        pltpu.async_copy(
            t3_hbm.at[0, pl.ds(tc0 * 8, 16)], stage_v.at[pb, pl.ds(0, 16)],
            isem.at[pb],
        )
        pltpu.async_copy(
            t3_hbm.at[1, pl.ds(tc0 * 8, 16)], stage_v.at[pb, pl.ds(16, 16)],
            isem.at[pb],
        )

    def p1_wait_load(i, pb):
        tc0 = p1_tc0(i)
        pltpu.make_async_copy(
            t3_hbm.at[0, pl.ds(tc0 * 8, 16)], stage_v.at[pb, pl.ds(0, 16)],
            isem.at[pb],
        ).wait()
        pltpu.make_async_copy(
            t3_hbm.at[1, pl.ds(tc0 * 8, 16)], stage_v.at[pb, pl.ds(16, 16)],
            isem.at[pb],
        ).wait()

    def p1_store_slice(i):
        tc0 = p1_tc0(i)
        return tbl_hbm.at[pl.ds(core_row0 + tc0 * 128, 256)]

    def p1_compute_fire(i, pb):
        zero16 = jnp.zeros((_L,), jnp.int32)
        for tcrel in range(2):
            rvec = rv[tcrel]

            def icg_body(g, c):
                icb = zero16 + g * _L
                vals = [
                    plsc.load_gather(stage_v.at[pb], [rvec, icb + k])
                    for k in range(_L)
                ]
                for k in range(_L):
                    ost_v[pb, tcrel * 128 + g * _L + k, :] = vals[k]
                return c

            lax.fori_loop(0, 128 // _L, icg_body, 0)
        pltpu.async_copy(ost_v.at[pb], p1_store_slice(i), psem.at[pb])

    def p1_wait_store(i, pb):
        pltpu.make_async_copy(ost_v.at[pb], p1_store_slice(i), psem.at[pb]).wait()

    p1_load(0, 0)

    def p1_step(i, c):
        pb = i % 2

        @pl.when(i + 1 < _P1_CHUNKS)
        def _():
            p1_load(i + 1, 1 - pb)

        p1_wait_load(i, pb)

        @pl.when(i >= 2)
        def _():
            p1_wait_store(i - 2, pb)

        p1_compute_fire(i, pb)
        return c

    lax.fori_loop(0, _P1_CHUNKS, p1_step, 0)
    p1_wait_store(_P1_CHUNKS - 2, (_P1_CHUNKS - 2) % 2)
    p1_wait_store(_P1_CHUNKS - 1, (_P1_CHUNKS - 1) % 2)
    plsc.subcore_barrier()

    def task_nb(t):
        n = t // _WINS_PER_N
        b0 = (t % _WINS_PER_N) * _W
        return n, b0

    def load_and_fire(t, b):
        n, b0 = task_nb(t)
        tr = n // 8
        ir = n % 8
        tc0 = b0 // 128
        pltpu.sync_copy(
            labels_hbm.at[pl.ds(tr * 128 + tc0, _W // 128), pl.ds(ir * 128, 128)],
            idx_v.at[b],
        )
        coff = jnp.zeros((_L,), jnp.int32) + core_row0

        def remap(r, c):
            for cc in range(128 // _L):
                v = idx_v[b, r, pl.ds(cc * _L, _L)]
                w = jnp.minimum(jnp.maximum(v, 0), _MAX_CLASSES)
                idx_v[b, r, pl.ds(cc * _L, _L)] = (
                    jnp.where(v == -1, _MAX_CLASSES, w) + coff
                )
            return c

        lax.fori_loop(0, _W // 128, remap, 0)

        def gfire(r, c):
            pltpu.async_copy(
                tbl_hbm.at[idx_v.at[b, r]],
                rows_v.at[b, pl.ds(r * 128, 128)],
                gsem.at[b],
            )
            return c

        lax.fori_loop(0, _W // 128, gfire, 0)

    def wait_gather(b):
        def gwait(r, c):
            pltpu.make_async_copy(
                tbl_hbm.at[idx_v.at[b, r]],
                rows_v.at[b, pl.ds(r * 128, 128)],
                gsem.at[b],
            ).wait()
            return c

        lax.fori_loop(0, _W // 128, gwait, 0)

    def transpose(b):
        iota = lax.iota(jnp.int32, _L)
        dconst = [jnp.full((_L,), d, jnp.int32) for d in range(_D)]

        def body(c, r_ids):
            ch = c // 8
            cl = c % 8
            vals = [
                plsc.load_gather(rows_v.at[b], [r_ids, dconst[d]])
                for d in range(_D)
            ]
            for d in range(_D):
                tr_v[d // 8, ch, d % 8, pl.ds(cl * _L, _L)] = vals[d]
            return r_ids + _L

        lax.fori_loop(0, _W // _L, body, iota, unroll=2)

    def fire_store(t):
        n, b0 = task_nb(t)
        tc0 = b0 // 128
        for h in range(2):
            pltpu.async_copy(
                tr_v.at[h], out_hbm.at[n, h, pl.ds(tc0, _W // 128)], ssem
            )

    def wait_store(t):
        n, b0 = task_nb(t)
        tc0 = b0 // 128
        for h in range(2):
            pltpu.make_async_copy(
                tr_v.at[h], out_hbm.at[n, h, pl.ds(tc0, _W // 128)], ssem
            ).wait()

    load_and_fire(t0, 0)

    def step(i, carry):
        t = t0 + i
        b = i % 2

        @pl.when(i < _PER_W - 1)
        def _():
            load_and_fire(t + 1, 1 - b)

        wait_gather(b)

        @pl.when(i >= 1)
        def _():
            wait_store(t - 1)

        transpose(b)
        fire_store(t)
        return carry

    lax.fori_loop(0, _PER_W, step, 0)
    wait_store(t0 + _PER_W - 1)


def kernel(labels, table):
    lt2 = (
        labels.T.reshape(_N // 8, 8, _B // 128, 128)
        .transpose(0, 2, 1, 3)
        .reshape((_N // 8) * (_B // 128), 8 * 128)
    )
    tp = jnp.pad(table, ((0, _VPAD - (_MAX_CLASSES + 1)), (0, 0)))
    t3 = (
        tp.T.reshape(2, 8, _NTC, 128)
        .transpose(0, 2, 1, 3)
        .reshape(2, _NTC * 8, 128)
    )
    out5, _ = _gather_kernel(lt2, t3)
    out_t = out5.transpose(0, 1, 3, 2, 4).reshape(_N, _D, _B)
    return jnp.transpose(out_t, (2, 0, 1))
